# baseline (speedup 1.0000x reference)
import jax
import jax.numpy as jnp
from jax import lax
from jax.experimental import pallas as pl
from jax.experimental.pallas import tpu as pltpu
from jax.experimental.pallas import tpu_sc as plsc

VOCAB = 1000000
EMB = 64
B = 4096
L = 50

_NC = 2
_NS = 16
_NW = _NC * _NS

_BPW = B // _NW
_LP = 56

_NBUF = 8
_LAG = 2

_PBLK = 128
_PFULL = VOCAB // _PBLK
_PTAIL = VOCAB - _PFULL * _PBLK
_PITER = _PFULL // _NW + 1
_PNBUF = 4
_PLAG = 1


def _pad_kernel(tin_hbm, out_hbm, buf_v, sem_g, sem_s):
    wid = lax.axis_index("s") * _NC + lax.axis_index("c")

    def fire_read(j):
        blk = wid + _NW * j
        pltpu.async_copy(
            tin_hbm.at[pl.ds(blk * _PBLK, _PBLK)],
            buf_v.at[j % _PNBUF, pl.ds(0, _PBLK), pl.ds(0, EMB)],
            sem_g,
        )

    def fire_write(j):
        blk = wid + _NW * j
        pltpu.async_copy(
            buf_v.at[j % _PNBUF], out_hbm.at[pl.ds(blk * _PBLK, _PBLK)], sem_s
        )

    def wait_read():
        pltpu.make_async_copy(
            tin_hbm.at[pl.ds(0, _PBLK)],
            buf_v.at[0, pl.ds(0, _PBLK), pl.ds(0, EMB)],
            sem_g,
        ).wait()

    def wait_write():
        pltpu.make_async_copy(
            buf_v.at[0], out_hbm.at[pl.ds(0, _PBLK)], sem_s
        ).wait()

    def in_range(j):
        return wid + _NW * j < _PFULL

    for j in range(_PNBUF):
        @pl.when(in_range(j))
        def _():
            fire_read(j)

    def step(j, carry):
        @pl.when(in_range(j))
        def _():
            wait_read()
            fire_write(j)

        @pl.when(j >= _PLAG)
        def _():
            @pl.when(in_range(j - _PLAG))
            def _():
                wait_write()

            @pl.when(in_range(j + _PNBUF - _PLAG))
            def _():
                fire_read(j + _PNBUF - _PLAG)

        return carry

    lax.fori_loop(0, _PITER, step, 0)
    @pl.when(in_range(_PITER - _PLAG))
    def _():
        wait_write()
    @pl.when(wid == _NW - 1)
    def _():
        r0 = _PFULL * _PBLK
        pltpu.sync_copy(
            tin_hbm.at[pl.ds(r0, _PTAIL)],
            buf_v.at[0, pl.ds(0, _PTAIL), pl.ds(0, EMB)],
        )
        pltpu.sync_copy(
            buf_v.at[0, pl.ds(0, _PTAIL)], out_hbm.at[pl.ds(r0, _PTAIL)]
        )


def _gather_kernel(table_hbm, x_hbm, out_hbm, idx_v, rows_v, sem_g, sem_s):
    wid = lax.axis_index("s") * _NC + lax.axis_index("c")
    b0 = wid * _BPW
    pltpu.sync_copy(x_hbm.at[pl.ds(b0, _BPW)], idx_v)

    def fire_gather(j):
        pltpu.async_copy(
            table_hbm.at[idx_v.at[j]], rows_v.at[j % _NBUF], sem_g
        )

    def fire_store(j):
        pltpu.async_copy(
            rows_v.at[j % _NBUF],
            out_hbm.at[b0 + j, pl.ds(0, L), pl.ds(0, EMB)],
            sem_s,
        )

    def wait_gather():
        pltpu.make_async_copy(
            table_hbm.at[pl.ds(0, L)], rows_v.at[0], sem_g
        ).wait()

    def wait_store():
        pltpu.make_async_copy(
            rows_v.at[0], out_hbm.at[0, pl.ds(0, L), pl.ds(0, EMB)], sem_s
        ).wait()

    for j in range(_NBUF):
        fire_gather(j)

    def step(j, carry):
        wait_gather()
        fire_store(j)
        @pl.when(j >= _LAG)
        def _():
            wait_store()

            @pl.when(j + _NBUF - _LAG < _BPW)
            def _():
                fire_gather(j + _NBUF - _LAG)

        return carry

    lax.fori_loop(0, _BPW, step, 0)
    for _ in range(_LAG):
        wait_store()


@jax.jit
def kernel(x, table):
    mesh = plsc.VectorSubcoreMesh(core_axis_name="c", subcore_axis_name="s")
    tpad = jnp.pad(table.T, ((0, 128 - EMB), (0, 0))).T
    tview = tpad.reshape(2 * VOCAB, EMB)
    idx2 = x.astype(jnp.int32) * 2
    out = pl.kernel(
        _gather_kernel,
        out_type=jax.ShapeDtypeStruct((B, _LP, 128), jnp.float32),
        mesh=mesh,
        scratch_types=[
            pltpu.VMEM((_BPW, L), jnp.int32),
            pltpu.VMEM((_NBUF, L, EMB), jnp.float32),
            pltpu.SemaphoreType.DMA,
            pltpu.SemaphoreType.DMA,
        ],
        compiler_params=pltpu.CompilerParams(use_tc_tiling_on_sc=False),
    )(tview, idx2)
    return out[:, :L, :EMB]
